# baseline (speedup 1.0000x reference)
import functools

import jax
import jax.numpy as jnp
from jax import lax
from jax.experimental import pallas as pl
from jax.experimental.pallas import tpu as pltpu
from jax.experimental.pallas import tpu_sc as plsc

VOCAB = 50257
HIDDEN = 768
MAX_LEN = 2048
BATCH = 4
SEQ = 2048

_INFO = plsc.get_sparse_core_info()
_NC = _INFO.num_cores
_NS = _INFO.num_subcores
_NW = _NC * _NS
_SPW = SEQ // _NW
_VECS = HIDDEN // 16
_CH = _SPW // 2
_HF = _CH // 2

_CHUNKS = [(h, b) for h in range(2) for b in range(BATCH)]


def _emb_body(idx_hbm, word_hbm, pos_hbm, out_hbm,
              idx_v, rows_a, rows_b, pos_v, ga, gb, sa, sb, psem):
    wid = lax.axis_index("s") * _NC + lax.axis_index("c")
    s0 = wid * _SPW

    pos_cp = pltpu.async_copy(pos_hbm.at[pl.ds(s0, _CH)], pos_v, psem)
    pltpu.sync_copy(idx_hbm.at[wid], idx_v)

    rows = (rows_a, rows_b)
    gsem = (ga, gb)
    ssem = (sa, sb)

    def start_gather(ci):
        h, b = _CHUNKS[ci]
        return pltpu.async_copy(
            word_hbm.at[idx_v.at[b, h]], rows[ci % 2], gsem[ci % 2])

    gathers = {0: start_gather(0)}
    stores = {}
    for ci, (h, b) in enumerate(_CHUNKS):
        buf = ci % 2
        r0 = s0 + h * _CH
        gathers[ci].wait()
        if ci == 0 or ci == 4:
            pos_cp.wait()

        def add_body(r, _, rv=rows[buf]):
            for j in range(_VECS):
                c = j * 16
                rv[r, pl.ds(c, 16)] = (
                    rv[r, pl.ds(c, 16)] + pos_v[r, pl.ds(c, 16)]
                )
            return _

        lax.fori_loop(0, _HF, add_body, 0)
        st0 = pltpu.async_copy(rows[buf].at[pl.ds(0, _HF)],
                               out_hbm.at[b, pl.ds(r0, _HF)], ssem[buf])
        lax.fori_loop(_HF, _CH, add_body, 0)
        if ci + 1 < len(_CHUNKS):
            if ci - 1 >= 0:
                stores[ci - 1][0].wait()
                stores[ci - 1][1].wait()
            gathers[ci + 1] = start_gather(ci + 1)
        if ci == 3:
            pos_cp = pltpu.async_copy(
                pos_hbm.at[pl.ds(s0 + _CH, _CH)], pos_v, psem)
        st1 = pltpu.async_copy(rows[buf].at[pl.ds(_HF, _HF)],
                               out_hbm.at[b, pl.ds(r0 + _HF, _HF)], ssem[buf])
        stores[ci] = (st0, st1)

    for st in stores[len(_CHUNKS) - 2] + stores[len(_CHUNKS) - 1]:
        st.wait()


@functools.partial(jax.jit, static_argnames=())
def _embed(indices, word_table, pos_table):
    idx4 = indices.reshape(BATCH, _NW, 2, _CH).transpose(1, 0, 2, 3)
    mesh = plsc.VectorSubcoreMesh(core_axis_name="c", subcore_axis_name="s")
    k = pl.kernel(
        _emb_body,
        out_type=jax.ShapeDtypeStruct((BATCH, SEQ, HIDDEN), jnp.float32),
        mesh=mesh,
        scratch_types=[
            pltpu.VMEM((BATCH, 2, _CH), jnp.int32),
            pltpu.VMEM((_CH, HIDDEN), jnp.float32),
            pltpu.VMEM((_CH, HIDDEN), jnp.float32),
            pltpu.VMEM((_CH, HIDDEN), jnp.float32),
            pltpu.SemaphoreType.DMA,
            pltpu.SemaphoreType.DMA,
            pltpu.SemaphoreType.DMA,
            pltpu.SemaphoreType.DMA,
            pltpu.SemaphoreType.DMA,
        ],
    )
    return k(idx4, word_table, pos_table)


def kernel(indices, word_table, pos_table):
    return _embed(indices, word_table, pos_table)


# speedup vs baseline: 1.0924x; 1.0924x over previous
import functools

import jax
import jax.numpy as jnp
from jax import lax
from jax.experimental import pallas as pl
from jax.experimental.pallas import tpu as pltpu
from jax.experimental.pallas import tpu_sc as plsc

VOCAB = 50257
HIDDEN = 768
MAX_LEN = 2048
BATCH = 4
SEQ = 2048

_INFO = plsc.get_sparse_core_info()
_NC = _INFO.num_cores
_NS = _INFO.num_subcores
_NW = _NC * _NS
_SPW = SEQ // _NW
_VECS = HIDDEN // 16


def _emb_body(idx_hbm, word_hbm, pos_hbm, out_hbm,
              idx_v, rows_v, pos_v, gsem, psem):
    wid = lax.axis_index("s") * _NC + lax.axis_index("c")
    s0 = wid * _SPW

    pos_cp = pltpu.async_copy(pos_hbm.at[pl.ds(s0, _SPW)], pos_v, psem)
    pltpu.sync_copy(idx_hbm.at[wid], idx_v)

    half = _SPW // 2
    for b in range(BATCH):
        pltpu.async_copy(word_hbm.at[idx_v.at[b]], rows_v, gsem).wait()
        if b == 0:
            pos_cp.wait()

        def add_body(r, _, rows_v=rows_v, pos_v=pos_v):
            for j in range(_VECS):
                c = j * 16
                rows_v[r, pl.ds(c, 16)] = (
                    rows_v[r, pl.ds(c, 16)] + pos_v[r, pl.ds(c, 16)]
                )
            return _

        lax.fori_loop(0, half, add_body, 0)
        st0 = pltpu.async_copy(rows_v.at[pl.ds(0, half)],
                               out_hbm.at[b, pl.ds(s0, half)], psem)
        lax.fori_loop(half, _SPW, add_body, 0)
        st0.wait()
        pltpu.sync_copy(rows_v.at[pl.ds(half, half)],
                        out_hbm.at[b, pl.ds(s0 + half, half)])


@functools.partial(jax.jit, static_argnames=())
def _embed(indices, word_table, pos_table):
    idx3 = indices.reshape(BATCH, _NW, _SPW).transpose(1, 0, 2)
    mesh = plsc.VectorSubcoreMesh(core_axis_name="c", subcore_axis_name="s")
    k = pl.kernel(
        _emb_body,
        out_type=jax.ShapeDtypeStruct((BATCH, SEQ, HIDDEN), jnp.float32),
        mesh=mesh,
        scratch_types=[
            pltpu.VMEM((BATCH, _SPW), jnp.int32),
            pltpu.VMEM((_SPW, HIDDEN), jnp.float32),
            pltpu.VMEM((_SPW, HIDDEN), jnp.float32),
            pltpu.SemaphoreType.DMA,
            pltpu.SemaphoreType.DMA,
        ],
    )
    return k(idx3, word_table, pos_table)


def kernel(indices, word_table, pos_table):
    return _embed(indices, word_table, pos_table)
